# baseline (speedup 1.0000x reference)
import functools

import jax
import jax.numpy as jnp
from jax import lax
from jax.experimental import pallas as pl
from jax.experimental.pallas import tpu as pltpu
from jax.experimental.pallas import tpu_sc as plsc

B, L, D = 1024, 512, 64
SCALE = 8.0
LANE = 16

_info = plsc.get_sparse_core_info()
NC = _info.num_cores
NS = _info.num_subcores
NW = NC * NS
SEQ_PER_W = B // NW
CH = 128
NBUF = L // CH

_mesh = plsc.VectorSubcoreMesh(core_axis_name="c", subcore_axis_name="s")


@functools.partial(
    pl.kernel,
    mesh=_mesh,
    out_type=jax.ShapeDtypeStruct((B, D // 8, L // 128, 8, 128), jnp.float32),
    scratch_types=[
        pltpu.VMEM((SEQ_PER_W, L), jnp.int32),
        pltpu.VMEM((D, L), jnp.float32),
    ]
    + [pltpu.VMEM((CH, D), jnp.float32) for _ in range(NBUF)]
    + [pltpu.VMEM((8, 8, 128), jnp.float32) for _ in range(NBUF)]
    + [pltpu.SemaphoreType.DMA for _ in range(2 * NBUF)],
    compiler_params=pltpu.CompilerParams(
        use_tc_tiling_on_sc=False, needs_layout_passes=False
    ),
)
def _emb_kernel(x_hbm, emb_hbm, post_hbm, out_hbm, idx_v, post_v, *bufs):
    rows = bufs[:NBUF]
    rt = bufs[NBUF:2 * NBUF]
    gsem = bufs[2 * NBUF:3 * NBUF]
    osem = bufs[3 * NBUF:]
    wid = lax.axis_index("s") * NC + lax.axis_index("c")
    base_seq = wid * SEQ_PER_W
    pltpu.sync_copy(x_hbm.at[pl.ds(base_seq, SEQ_PER_W)], idx_v)
    pltpu.sync_copy(post_hbm, post_v)

    def gather(q, quarter, b):
        idx_ref = idx_v.at[q, pl.ds(quarter * CH, CH)]
        return pltpu.make_async_copy(emb_hbm.at[idx_ref], rows[b], gsem[b])

    def writeout(q, quarter, b):
        return pltpu.make_async_copy(
            rt[b], out_hbm.at[base_seq + q, :, quarter], osem[b]
        )

    for b in range(NBUF - 1):
        gather(0, b, b).start()

    def seq_body(g, carry):
        for b in range(NBUF):
            gather(g, b, b).wait()

            @pl.when(g >= 1)
            def _drain_rt():
                writeout(g - 1, b, b).wait()

            @plsc.parallel_loop(0, CH // LANE)
            def _lig(lig):
                ridx = lig * LANE + lax.iota(jnp.int32, LANE)
                loff = b * CH + lig * LANE
                for d in range(D):
                    cidx = jnp.full((LANE,), d, jnp.int32)
                    v = plsc.load_gather(rows[b], [ridx, cidx])
                    p = post_v[d, pl.ds(loff, LANE)]
                    rt[b][d // 8, d % 8, pl.ds(lig * LANE, LANE)] = v * SCALE + p

            writeout(g, b, b).start()

            b3 = (b + NBUF - 1) % NBUF
            if b == 0:
                gather(g, NBUF - 1, b3).start()
            else:
                @pl.when(g < SEQ_PER_W - 1)
                def _pref():
                    gather(g + 1, b - 1, b3).start()
        return carry

    lax.fori_loop(0, SEQ_PER_W, seq_body, 0)

    for b in range(NBUF):
        writeout(SEQ_PER_W - 1, b, b).wait()


def kernel(x, emb_table, pos_table):
    r5 = _emb_kernel(x, emb_table, pos_table.T)
    return r5.transpose(0, 2, 4, 1, 3).reshape(B, L, D)


# speedup vs baseline: 1.1647x; 1.1647x over previous
import functools

import jax
import jax.numpy as jnp
from jax import lax
from jax.experimental import pallas as pl
from jax.experimental.pallas import tpu as pltpu
from jax.experimental.pallas import tpu_sc as plsc

B, L, D = 1024, 512, 64
SCALE = 8.0
LANE = 16

_info = plsc.get_sparse_core_info()
NC = _info.num_cores
NS = _info.num_subcores
NW = NC * NS
SEQ_PER_W = B // NW
CH = 128
NBUF = L // CH

_mesh = plsc.VectorSubcoreMesh(core_axis_name="c", subcore_axis_name="s")


@functools.partial(
    pl.kernel,
    mesh=_mesh,
    out_type=jax.ShapeDtypeStruct((B, L, D), jnp.float32),
    scratch_types=[
        pltpu.VMEM((SEQ_PER_W, L), jnp.int32),
        pltpu.VMEM((L, D), jnp.float32),
    ]
    + [pltpu.VMEM((CH, D), jnp.float32) for _ in range(NBUF)]
    + [pltpu.SemaphoreType.DMA for _ in range(2 * NBUF)],
    compiler_params=pltpu.CompilerParams(use_tc_tiling_on_sc=False),
)
def _emb_kernel(x_hbm, emb_hbm, pos_hbm, out_hbm, idx_v, pos_v, *bufs):
    rows = bufs[:NBUF]
    gsem = bufs[NBUF:2 * NBUF]
    osem = bufs[2 * NBUF:]
    wid = lax.axis_index("s") * NC + lax.axis_index("c")
    base_seq = wid * SEQ_PER_W
    pltpu.sync_copy(x_hbm.at[pl.ds(base_seq, SEQ_PER_W)], idx_v)
    pltpu.sync_copy(pos_hbm, pos_v)

    def gather(q, quarter, b):
        idx_ref = idx_v.at[q, pl.ds(quarter * CH, CH)]
        return pltpu.make_async_copy(emb_hbm.at[idx_ref], rows[b], gsem[b])

    def writeout(q, quarter, b):
        return pltpu.make_async_copy(
            rows[b], out_hbm.at[base_seq + q, pl.ds(quarter * CH, CH)], osem[b]
        )

    for b in range(NBUF - 1):
        gather(0, b, b).start()

    def seq_body(g, carry):
        for b in range(NBUF):
            gather(g, b, b).wait()

            @plsc.parallel_loop(0, CH, unroll=4)
            def _row(r):
                for d in range(D // LANE):
                    sl = pl.ds(d * LANE, LANE)
                    rows[b][r, sl] = rows[b][r, sl] * SCALE + pos_v[b * CH + r, sl]

            writeout(g, b, b).start()

            b3 = (b + NBUF - 1) % NBUF
            if b == 0:
                @pl.when(g >= 1)
                def _drain():
                    writeout(g - 1, NBUF - 1, b3).wait()

                gather(g, NBUF - 1, b3).start()
            else:
                @pl.when(g < SEQ_PER_W - 1)
                def _pref():
                    writeout(g, b - 1, b3).wait()
                    gather(g + 1, b - 1, b3).start()
        return carry

    lax.fori_loop(0, SEQ_PER_W, seq_body, 0)

    for b in range(NBUF):
        writeout(SEQ_PER_W - 1, b, b).wait()


def kernel(x, emb_table, pos_table):
    return _emb_kernel(x, emb_table, pos_table)
